# baseline (speedup 1.0000x reference)
import jax
import jax.numpy as jnp
from jax import lax
from jax.experimental import pallas as pl
from jax.experimental.pallas import tpu as pltpu
from jax.experimental.pallas import tpu_sc as plsc

N_NODES = 10000
N_PAD = 10240
D = 256
NQ = 4
HQ = 64
C = 125
NBUF = 4

NC = 2
NS = 16

_ROWS_PER_TILE = N_PAD // NS
_ZB = 160


def _sc_mesh():
    return plsc.VectorSubcoreMesh(
        core_axis_name="c", subcore_axis_name="s", num_cores=NC, num_subcores=NS
    )


def _deg_body(dst3_hbm, out_hbm, idx_v, ones_v, zero_v, acc, ssem):
    c = lax.axis_index("c")
    s = lax.axis_index("s")
    per_tile = dst3_hbm.shape[1]

    def fill(i, _):
        ones_v[i, :] = jnp.full((16,), 1.0, jnp.float32)
        return 0

    lax.fori_loop(0, C, fill, 0)

    def zfill(i, _):
        zero_v[i, :] = jnp.zeros((16,), jnp.float32)
        return 0

    lax.fori_loop(0, _ROWS_PER_TILE, zfill, 0)

    pltpu.sync_copy(zero_v, acc.at[pl.ds(s * _ROWS_PER_TILE, _ROWS_PER_TILE)])
    pltpu.sync_copy(dst3_hbm.at[s], idx_v)
    plsc.subcore_barrier()

    def body(j, _):
        pltpu.async_copy(ones_v, acc.at[idx_v.at[j]], ssem, add=True)
        return 0

    lax.fori_loop(0, per_tile, body, 0)

    def drain(j, _):
        pltpu.make_async_copy(ones_v, acc.at[idx_v.at[j]], ssem).wait()
        return 0

    lax.fori_loop(0, per_tile, drain, 0)
    plsc.subcore_barrier()

    @pl.when(c == 0)
    def _():
        pltpu.sync_copy(
            acc.at[pl.ds(s * _ROWS_PER_TILE, _ROWS_PER_TILE)],
            out_hbm.at[pl.ds(s * _ROWS_PER_TILE, _ROWS_PER_TILE)],
        )


def _deg_counts(dst3):
    per_tile = dst3.shape[1]
    k = pl.kernel(
        _deg_body,
        out_type=jax.ShapeDtypeStruct((N_PAD, 16), jnp.float32),
        mesh=_sc_mesh(),
        compiler_params=pltpu.CompilerParams(use_tc_tiling_on_sc=False),
        scratch_types=[
            pltpu.VMEM((per_tile, C), jnp.int32),
            pltpu.VMEM((C, 16), jnp.float32),
            pltpu.VMEM((_ROWS_PER_TILE, 16), jnp.float32),
            pltpu.VMEM_SHARED((N_PAD, 16), jnp.float32),
            pltpu.SemaphoreType.DMA,
        ],
    )
    return k(dst3)


def _hop_body(m_hbm, src3_hbm, dst3_hbm, out_hbm, isrc_v, idst_v, rows_v, zb_v, acc, gsem, ssem):
    c = lax.axis_index("c")
    s = lax.axis_index("s")
    per_tile = src3_hbm.shape[1]

    def zrow(i, _):
        def zcol(j, _):
            zb_v[i, pl.ds(j * 16, 16)] = jnp.zeros((16,), jnp.float32)
            return 0

        lax.fori_loop(0, HQ // 16, zcol, 0)
        return 0

    lax.fori_loop(0, _ZB, zrow, 0)

    pltpu.sync_copy(src3_hbm.at[s], isrc_v)
    pltpu.sync_copy(dst3_hbm.at[s], idst_v)

    for p in range(2):
        q = 2 * c + p
        for i in range(_ROWS_PER_TILE // _ZB):
            pltpu.sync_copy(zb_v, acc.at[pl.ds(s * _ROWS_PER_TILE + i * _ZB, _ZB)])
        plsc.subcore_barrier()

        for b in range(NBUF):
            pltpu.async_copy(m_hbm.at[q].at[isrc_v.at[b]], rows_v.at[b], gsem.at[b])

        def group(g, _):
            for b in range(NBUF):
                ch = g * NBUF + b
                pltpu.make_async_copy(
                    m_hbm.at[q].at[isrc_v.at[ch]], rows_v.at[b], gsem.at[b]
                ).wait()
                pltpu.async_copy(
                    rows_v.at[b], acc.at[idst_v.at[ch]], ssem.at[b], add=True
                )
            for b in range(NBUF):
                ch = g * NBUF + b
                pltpu.make_async_copy(
                    rows_v.at[b], acc.at[idst_v.at[ch]], ssem.at[b]
                ).wait()
                nxt = ch + NBUF

                @pl.when(nxt < per_tile)
                def _():
                    pltpu.async_copy(
                        m_hbm.at[q].at[isrc_v.at[nxt]], rows_v.at[b], gsem.at[b]
                    )
            return 0

        lax.fori_loop(0, per_tile // NBUF, group, 0)
        plsc.subcore_barrier()

        pltpu.sync_copy(
            acc.at[pl.ds(s * _ROWS_PER_TILE, _ROWS_PER_TILE)],
            out_hbm.at[q].at[pl.ds(s * _ROWS_PER_TILE, _ROWS_PER_TILE)],
        )


def _hop(m_q, src3, dst3):
    per_tile = src3.shape[1]
    k = pl.kernel(
        _hop_body,
        out_type=jax.ShapeDtypeStruct((NQ, N_PAD, HQ), jnp.float32),
        mesh=_sc_mesh(),
        compiler_params=pltpu.CompilerParams(use_tc_tiling_on_sc=False),
        scratch_types=[
            pltpu.VMEM((per_tile, C), jnp.int32),
            pltpu.VMEM((per_tile, C), jnp.int32),
            pltpu.VMEM((NBUF, C, HQ), jnp.float32),
            pltpu.VMEM((_ZB, HQ), jnp.float32),
            pltpu.VMEM_SHARED((N_PAD, HQ), jnp.float32),
            pltpu.SemaphoreType.DMA((NBUF,)),
            pltpu.SemaphoreType.DMA((NBUF,)),
        ],
    )
    return k(m_q, src3, dst3)


_RB = 1000

_BLKQ = pl.BlockSpec((NQ, _RB, HQ), lambda i: (0, i, 0))
_BLKF = pl.BlockSpec((_RB, D), lambda i: (i, 0))


def _split_q(o_ref, res):
    for q in range(NQ):
        o_ref[q] = res[:, q * HQ:(q + 1) * HQ]


def _fc_in_kernel(h_ref, w_ref, b_ref, o_ref, oq_ref):
    res = jnp.dot(h_ref[...], w_ref[...], preferred_element_type=jnp.float32)
    res = res + b_ref[...]
    o_ref[...] = res
    _split_q(oq_ref, res)


def _fc_in(h, w, b):
    return pl.pallas_call(
        _fc_in_kernel,
        grid=(N_NODES // _RB,),
        in_specs=[
            _BLKF,
            pl.BlockSpec((D, D), lambda i: (0, 0)),
            pl.BlockSpec((1, D), lambda i: (0, 0)),
        ],
        out_specs=[_BLKF, _BLKQ],
        out_shape=[
            jax.ShapeDtypeStruct((N_NODES, D), jnp.float32),
            jax.ShapeDtypeStruct((NQ, N_NODES, HQ), jnp.float32),
        ],
    )(h, w, b.reshape(1, D))


def _gru_kernel(x_ref, s_ref, w_ref, u_ref, b_ref, o_ref):
    gi = jnp.dot(x_ref[...], w_ref[...], preferred_element_type=jnp.float32)
    gi = gi + b_ref[...]
    gh = jnp.dot(s_ref[...], u_ref[...], preferred_element_type=jnp.float32)
    z = jax.nn.sigmoid(gi[:, :D] + gh[:, :D])
    r = jax.nn.sigmoid(gi[:, D:2 * D] + gh[:, D:2 * D])
    nn = jnp.tanh(gi[:, 2 * D:] + r * gh[:, 2 * D:])
    o_ref[...] = (1.0 - z) * nn + z * s_ref[...]


def _gru_step(x, s, w, u, b):
    return pl.pallas_call(
        _gru_kernel,
        grid=(N_NODES // _RB,),
        in_specs=[
            _BLKF,
            _BLKF,
            pl.BlockSpec((D, 3 * D), lambda i: (0, 0)),
            pl.BlockSpec((D, 3 * D), lambda i: (0, 0)),
            pl.BlockSpec((1, 3 * D), lambda i: (0, 0)),
        ],
        out_specs=_BLKF,
        out_shape=jax.ShapeDtypeStruct((N_NODES, D), jnp.float32),
    )(x, s, w, u, b.reshape(1, 3 * D))


def _gru_first_kernel(x_ref, w_ref, b_ref, o_ref):
    gi = jnp.dot(x_ref[...], w_ref[...], preferred_element_type=jnp.float32)
    gi = gi + b_ref[...]
    z = jax.nn.sigmoid(gi[:, :D])
    nn = jnp.tanh(gi[:, 2 * D:])
    o_ref[...] = (1.0 - z) * nn


def _gru_first(x, w, b):
    return pl.pallas_call(
        _gru_first_kernel,
        grid=(N_NODES // _RB,),
        in_specs=[
            _BLKF,
            pl.BlockSpec((D, 3 * D), lambda i: (0, 0)),
            pl.BlockSpec((1, 3 * D), lambda i: (0, 0)),
        ],
        out_specs=_BLKF,
        out_shape=jax.ShapeDtypeStruct((N_NODES, D), jnp.float32),
    )(x, w, b.reshape(1, 3 * D))


def _scale_kernel(agg_ref, d_ref, o_ref, oq_ref):
    inv = 1.0 / jnp.maximum(d_ref[:, 0:1], 1.0)
    parts = [agg_ref[q] * inv for q in range(NQ)]
    o_ref[...] = jnp.concatenate(parts, axis=1)
    for q in range(NQ):
        oq_ref[q] = parts[q]


def _scale(agg, dcnt):
    return pl.pallas_call(
        _scale_kernel,
        grid=(N_NODES // _RB,),
        in_specs=[
            _BLKQ,
            pl.BlockSpec((_RB, 16), lambda i: (i, 0)),
        ],
        out_specs=[_BLKF, _BLKQ],
        out_shape=[
            jax.ShapeDtypeStruct((N_NODES, D), jnp.float32),
            jax.ShapeDtypeStruct((NQ, N_NODES, HQ), jnp.float32),
        ],
    )(agg, dcnt)


def _elu(x):
    return jnp.where(x > 0, x, jnp.exp(x) - 1.0)


def _tail_kernel(s1, s2, s3, s4, h0_ref, dw_ref, db_ref, ym_ref, ymq_ref, loss_ref):
    ys = [_elu(sr[...]) for sr in (s1, s2, s3, s4)]
    ym = (ys[0] + ys[1] + ys[2] + ys[3]) * 0.25
    recon = jnp.dot(ym, dw_ref[...], preferred_element_type=jnp.float32)
    recon = recon + db_ref[...]
    diff = recon - h0_ref[...]

    @pl.when(pl.program_id(0) == 0)
    def _():
        loss_ref[...] = jnp.zeros((1, 1), jnp.float32)

    loss_ref[...] += jnp.sum(diff * diff).reshape(1, 1)
    ym_ref[...] = ym
    _split_q(ymq_ref, ym)


def _layer_tail(s_list, h0, dec_w, dec_b):
    return pl.pallas_call(
        _tail_kernel,
        grid=(N_NODES // _RB,),
        in_specs=[
            _BLKF, _BLKF, _BLKF, _BLKF,
            _BLKF,
            pl.BlockSpec((D, D), lambda i: (0, 0)),
            pl.BlockSpec((1, D), lambda i: (0, 0)),
        ],
        out_specs=[
            _BLKF,
            _BLKQ,
            pl.BlockSpec((1, 1), lambda i: (0, 0)),
        ],
        out_shape=[
            jax.ShapeDtypeStruct((N_NODES, D), jnp.float32),
            jax.ShapeDtypeStruct((NQ, N_NODES, HQ), jnp.float32),
            jax.ShapeDtypeStruct((1, 1), jnp.float32),
        ],
    )(*s_list, h0, dec_w, dec_b.reshape(1, D))


def _fc_out_kernel(s1, s2, s3, s4, w_ref, b_ref, o_ref):
    for t, sr in enumerate((s1, s2, s3, s4)):
        y = _elu(sr[...])
        logits = jnp.dot(y, w_ref[...], preferred_element_type=jnp.float32)
        logits = logits + b_ref[...]
        m = jnp.max(logits, axis=1, keepdims=True)
        p = jnp.exp(logits - m)
        o_ref[t] = p / jnp.sum(p, axis=1, keepdims=True)


def _fc_out(s_list, w, b):
    dout = w.shape[1]
    return pl.pallas_call(
        _fc_out_kernel,
        grid=(N_NODES // _RB,),
        in_specs=[
            _BLKF, _BLKF, _BLKF, _BLKF,
            pl.BlockSpec((D, dout), lambda i: (0, 0)),
            pl.BlockSpec((1, dout), lambda i: (0, 0)),
        ],
        out_specs=pl.BlockSpec((4, _RB, dout), lambda i: (0, i, 0)),
        out_shape=jax.ShapeDtypeStruct((4, N_NODES, dout), jnp.float32),
    )(*s_list, w, b.reshape(1, dout))


def kernel(h, edge_index, params):
    n, d_in = h.shape
    e = edge_index.shape[1]
    assert n == N_NODES and d_in == D and e % (C * NS) == 0 and (e // (C * NS)) % NBUF == 0

    src3 = edge_index[0].reshape(NS, e // C // NS, C).astype(jnp.int32)
    dst3 = edge_index[1].reshape(NS, e // C // NS, C).astype(jnp.int32)

    dcnt = _deg_counts(dst3)
    x, xq = _fc_in(h, params['fc_in_w'], params['fc_in_b'])

    loss = jnp.float32(0.0)
    out = None
    walk_len = 4
    for li, p in enumerate(params['layers']):
        mq = xq
        states = []
        s = _gru_first(x, p['W'], p['b'])
        for _ in range(walk_len):
            agg = _hop(mq, src3, dst3)
            m, mq = _scale(agg, dcnt)
            s = _gru_step(m, s, p['W'], p['U'], p['b'])
            states.append(s)
        ymean, ymq, losssum = _layer_tail(states, h, p['dec_w'], p['dec_b'])
        loss = loss + 0.05 * (losssum[0, 0] / (n * d_in))
        x, xq = ymean, ymq
        if li == len(params['layers']) - 1:
            out = _fc_out(states, params['fc_out_w'], params['fc_out_b'])
    return out, loss


# speedup vs baseline: 1.1089x; 1.1089x over previous
import jax
import jax.numpy as jnp
from jax import lax
from jax.experimental import pallas as pl
from jax.experimental.pallas import tpu as pltpu
from jax.experimental.pallas import tpu_sc as plsc

N_NODES = 10000
N_PAD = 10240
D = 256
NQ = 4
HQ = 64
C = 125
NBUF = 5

NC = 2
NS = 16

_ROWS_PER_TILE = N_PAD // NS
_ZB = 160


def _sc_mesh():
    return plsc.VectorSubcoreMesh(
        core_axis_name="c", subcore_axis_name="s", num_cores=NC, num_subcores=NS
    )


def _deg_body(dst3_hbm, out_hbm, idx_v, ones_v, zero_v, acc, ssem):
    c = lax.axis_index("c")
    s = lax.axis_index("s")
    per_tile = dst3_hbm.shape[1]

    def fill(i, _):
        ones_v[i, :] = jnp.full((16,), 1.0, jnp.float32)
        return 0

    lax.fori_loop(0, C, fill, 0)

    def zfill(i, _):
        zero_v[i, :] = jnp.zeros((16,), jnp.float32)
        return 0

    lax.fori_loop(0, _ROWS_PER_TILE, zfill, 0)

    pltpu.sync_copy(zero_v, acc.at[pl.ds(s * _ROWS_PER_TILE, _ROWS_PER_TILE)])
    pltpu.sync_copy(dst3_hbm.at[s], idx_v)
    plsc.subcore_barrier()

    def body(j, _):
        pltpu.async_copy(ones_v, acc.at[idx_v.at[j]], ssem, add=True)
        return 0

    lax.fori_loop(0, per_tile, body, 0)

    def drain(j, _):
        pltpu.make_async_copy(ones_v, acc.at[idx_v.at[j]], ssem).wait()
        return 0

    lax.fori_loop(0, per_tile, drain, 0)
    plsc.subcore_barrier()

    @pl.when(c == 0)
    def _():
        pltpu.sync_copy(
            acc.at[pl.ds(s * _ROWS_PER_TILE, _ROWS_PER_TILE)],
            out_hbm.at[pl.ds(s * _ROWS_PER_TILE, _ROWS_PER_TILE)],
        )


def _deg_counts(dst3):
    per_tile = dst3.shape[1]
    k = pl.kernel(
        _deg_body,
        out_type=jax.ShapeDtypeStruct((N_PAD, 16), jnp.float32),
        mesh=_sc_mesh(),
        compiler_params=pltpu.CompilerParams(use_tc_tiling_on_sc=False),
        scratch_types=[
            pltpu.VMEM((per_tile, C), jnp.int32),
            pltpu.VMEM((C, 16), jnp.float32),
            pltpu.VMEM((_ROWS_PER_TILE, 16), jnp.float32),
            pltpu.VMEM_SHARED((N_PAD, 16), jnp.float32),
            pltpu.SemaphoreType.DMA,
        ],
    )
    return k(dst3)


def _hop_body(m_hbm, src3_hbm, dst3_hbm, out_hbm, isrc_v, idst_v, rows_v, zb_v, acc, gsem):
    c = lax.axis_index("c")
    s = lax.axis_index("s")
    per_tile = src3_hbm.shape[1]

    def zrow(i, _):
        def zcol(j, _):
            zb_v[i, pl.ds(j * 16, 16)] = jnp.zeros((16,), jnp.float32)
            return 0

        lax.fori_loop(0, HQ // 16, zcol, 0)
        return 0

    lax.fori_loop(0, _ZB, zrow, 0)

    pltpu.sync_copy(src3_hbm.at[s], isrc_v)
    pltpu.sync_copy(dst3_hbm.at[s], idst_v)

    def prime(q):
        for b in range(NBUF):
            pltpu.async_copy(m_hbm.at[q].at[isrc_v.at[b]], rows_v.at[b], gsem.at[b])

    prime(2 * c)

    for p in range(2):
        q = 2 * c + p
        for i in range(_ROWS_PER_TILE // _ZB):
            pltpu.sync_copy(zb_v, acc.at[pl.ds(s * _ROWS_PER_TILE + i * _ZB, _ZB)])
        plsc.subcore_barrier()

        def group(g, _):
            for b in range(NBUF):
                ch = g * NBUF + b
                pltpu.make_async_copy(
                    m_hbm.at[q].at[isrc_v.at[ch]], rows_v.at[b], gsem.at[b]
                ).wait()
                pltpu.sync_copy(rows_v.at[b], acc.at[idst_v.at[ch]], add=True)
                nxt = ch + NBUF

                @pl.when(nxt < per_tile)
                def _():
                    pltpu.async_copy(
                        m_hbm.at[q].at[isrc_v.at[nxt]], rows_v.at[b], gsem.at[b]
                    )
            return 0

        lax.fori_loop(0, per_tile // NBUF, group, 0)
        if p == 0:
            prime(2 * c + 1)
        plsc.subcore_barrier()

        pltpu.sync_copy(
            acc.at[pl.ds(s * _ROWS_PER_TILE, _ROWS_PER_TILE)],
            out_hbm.at[q].at[pl.ds(s * _ROWS_PER_TILE, _ROWS_PER_TILE)],
        )


def _hop(m_q, src3, dst3):
    per_tile = src3.shape[1]
    k = pl.kernel(
        _hop_body,
        out_type=jax.ShapeDtypeStruct((NQ, N_PAD, HQ), jnp.float32),
        mesh=_sc_mesh(),
        compiler_params=pltpu.CompilerParams(use_tc_tiling_on_sc=False),
        scratch_types=[
            pltpu.VMEM((per_tile, C), jnp.int32),
            pltpu.VMEM((per_tile, C), jnp.int32),
            pltpu.VMEM((NBUF, C, HQ), jnp.float32),
            pltpu.VMEM((_ZB, HQ), jnp.float32),
            pltpu.VMEM_SHARED((N_PAD, HQ), jnp.float32),
            pltpu.SemaphoreType.DMA((NBUF,)),
        ],
    )
    return k(m_q, src3, dst3)


_RB = 1000

_BLKQ = pl.BlockSpec((NQ, _RB, HQ), lambda i: (0, i, 0))
_BLKF = pl.BlockSpec((_RB, D), lambda i: (i, 0))


def _split_q(o_ref, res):
    for q in range(NQ):
        o_ref[q] = res[:, q * HQ:(q + 1) * HQ]


def _fc_in_kernel(h_ref, w_ref, b_ref, o_ref, oq_ref):
    res = jnp.dot(h_ref[...], w_ref[...], preferred_element_type=jnp.float32)
    res = res + b_ref[...]
    o_ref[...] = res
    _split_q(oq_ref, res)


def _fc_in(h, w, b):
    return pl.pallas_call(
        _fc_in_kernel,
        grid=(N_NODES // _RB,),
        in_specs=[
            _BLKF,
            pl.BlockSpec((D, D), lambda i: (0, 0)),
            pl.BlockSpec((1, D), lambda i: (0, 0)),
        ],
        out_specs=[_BLKF, _BLKQ],
        out_shape=[
            jax.ShapeDtypeStruct((N_NODES, D), jnp.float32),
            jax.ShapeDtypeStruct((NQ, N_NODES, HQ), jnp.float32),
        ],
    )(h, w, b.reshape(1, D))


def _gru_kernel(x_ref, s_ref, w_ref, u_ref, b_ref, o_ref):
    gi = jnp.dot(x_ref[...], w_ref[...], preferred_element_type=jnp.float32)
    gi = gi + b_ref[...]
    gh = jnp.dot(s_ref[...], u_ref[...], preferred_element_type=jnp.float32)
    z = jax.nn.sigmoid(gi[:, :D] + gh[:, :D])
    r = jax.nn.sigmoid(gi[:, D:2 * D] + gh[:, D:2 * D])
    nn = jnp.tanh(gi[:, 2 * D:] + r * gh[:, 2 * D:])
    o_ref[...] = (1.0 - z) * nn + z * s_ref[...]


def _gru_step(x, s, w, u, b):
    return pl.pallas_call(
        _gru_kernel,
        grid=(N_NODES // _RB,),
        in_specs=[
            _BLKF,
            _BLKF,
            pl.BlockSpec((D, 3 * D), lambda i: (0, 0)),
            pl.BlockSpec((D, 3 * D), lambda i: (0, 0)),
            pl.BlockSpec((1, 3 * D), lambda i: (0, 0)),
        ],
        out_specs=_BLKF,
        out_shape=jax.ShapeDtypeStruct((N_NODES, D), jnp.float32),
    )(x, s, w, u, b.reshape(1, 3 * D))


def _gru_first_kernel(x_ref, w_ref, b_ref, o_ref):
    gi = jnp.dot(x_ref[...], w_ref[...], preferred_element_type=jnp.float32)
    gi = gi + b_ref[...]
    z = jax.nn.sigmoid(gi[:, :D])
    nn = jnp.tanh(gi[:, 2 * D:])
    o_ref[...] = (1.0 - z) * nn


def _gru_first(x, w, b):
    return pl.pallas_call(
        _gru_first_kernel,
        grid=(N_NODES // _RB,),
        in_specs=[
            _BLKF,
            pl.BlockSpec((D, 3 * D), lambda i: (0, 0)),
            pl.BlockSpec((1, 3 * D), lambda i: (0, 0)),
        ],
        out_specs=_BLKF,
        out_shape=jax.ShapeDtypeStruct((N_NODES, D), jnp.float32),
    )(x, w, b.reshape(1, 3 * D))


def _scale_kernel(agg_ref, d_ref, o_ref, oq_ref):
    inv = 1.0 / jnp.maximum(d_ref[:, 0:1], 1.0)
    parts = [agg_ref[q] * inv for q in range(NQ)]
    o_ref[...] = jnp.concatenate(parts, axis=1)
    for q in range(NQ):
        oq_ref[q] = parts[q]


def _scale(agg, dcnt):
    return pl.pallas_call(
        _scale_kernel,
        grid=(N_NODES // _RB,),
        in_specs=[
            _BLKQ,
            pl.BlockSpec((_RB, 16), lambda i: (i, 0)),
        ],
        out_specs=[_BLKF, _BLKQ],
        out_shape=[
            jax.ShapeDtypeStruct((N_NODES, D), jnp.float32),
            jax.ShapeDtypeStruct((NQ, N_NODES, HQ), jnp.float32),
        ],
    )(agg, dcnt)


def _elu(x):
    return jnp.where(x > 0, x, jnp.exp(x) - 1.0)


def _tail_kernel(s1, s2, s3, s4, h0_ref, dw_ref, db_ref, ym_ref, ymq_ref, loss_ref):
    ys = [_elu(sr[...]) for sr in (s1, s2, s3, s4)]
    ym = (ys[0] + ys[1] + ys[2] + ys[3]) * 0.25
    recon = jnp.dot(ym, dw_ref[...], preferred_element_type=jnp.float32)
    recon = recon + db_ref[...]
    diff = recon - h0_ref[...]

    @pl.when(pl.program_id(0) == 0)
    def _():
        loss_ref[...] = jnp.zeros((1, 1), jnp.float32)

    loss_ref[...] += jnp.sum(diff * diff).reshape(1, 1)
    ym_ref[...] = ym
    _split_q(ymq_ref, ym)


def _layer_tail(s_list, h0, dec_w, dec_b):
    return pl.pallas_call(
        _tail_kernel,
        grid=(N_NODES // _RB,),
        in_specs=[
            _BLKF, _BLKF, _BLKF, _BLKF,
            _BLKF,
            pl.BlockSpec((D, D), lambda i: (0, 0)),
            pl.BlockSpec((1, D), lambda i: (0, 0)),
        ],
        out_specs=[
            _BLKF,
            _BLKQ,
            pl.BlockSpec((1, 1), lambda i: (0, 0)),
        ],
        out_shape=[
            jax.ShapeDtypeStruct((N_NODES, D), jnp.float32),
            jax.ShapeDtypeStruct((NQ, N_NODES, HQ), jnp.float32),
            jax.ShapeDtypeStruct((1, 1), jnp.float32),
        ],
    )(*s_list, h0, dec_w, dec_b.reshape(1, D))


def _fc_out_kernel(s1, s2, s3, s4, w_ref, b_ref, o_ref):
    for t, sr in enumerate((s1, s2, s3, s4)):
        y = _elu(sr[...])
        logits = jnp.dot(y, w_ref[...], preferred_element_type=jnp.float32)
        logits = logits + b_ref[...]
        m = jnp.max(logits, axis=1, keepdims=True)
        p = jnp.exp(logits - m)
        o_ref[t] = p / jnp.sum(p, axis=1, keepdims=True)


def _fc_out(s_list, w, b):
    dout = w.shape[1]
    return pl.pallas_call(
        _fc_out_kernel,
        grid=(N_NODES // _RB,),
        in_specs=[
            _BLKF, _BLKF, _BLKF, _BLKF,
            pl.BlockSpec((D, dout), lambda i: (0, 0)),
            pl.BlockSpec((1, dout), lambda i: (0, 0)),
        ],
        out_specs=pl.BlockSpec((4, _RB, dout), lambda i: (0, i, 0)),
        out_shape=jax.ShapeDtypeStruct((4, N_NODES, dout), jnp.float32),
    )(*s_list, w, b.reshape(1, dout))


def kernel(h, edge_index, params):
    n, d_in = h.shape
    e = edge_index.shape[1]
    assert n == N_NODES and d_in == D and e % (C * NS) == 0 and (e // (C * NS)) % NBUF == 0

    src3 = edge_index[0].reshape(NS, e // C // NS, C).astype(jnp.int32)
    dst3 = edge_index[1].reshape(NS, e // C // NS, C).astype(jnp.int32)

    dcnt = _deg_counts(dst3)
    x, xq = _fc_in(h, params['fc_in_w'], params['fc_in_b'])

    loss = jnp.float32(0.0)
    out = None
    walk_len = 4
    for li, p in enumerate(params['layers']):
        mq = xq
        states = []
        s = _gru_first(x, p['W'], p['b'])
        for _ in range(walk_len):
            agg = _hop(mq, src3, dst3)
            m, mq = _scale(agg, dcnt)
            s = _gru_step(m, s, p['W'], p['U'], p['b'])
            states.append(s)
        ymean, ymq, losssum = _layer_tail(states, h, p['dec_w'], p['dec_b'])
        loss = loss + 0.05 * (losssum[0, 0] / (n * d_in))
        x, xq = ymean, ymq
        if li == len(params['layers']) - 1:
            out = _fc_out(states, params['fc_out_w'], params['fc_out_b'])
    return out, loss


# speedup vs baseline: 1.1103x; 1.0012x over previous
import jax
import jax.numpy as jnp
from jax import lax
from jax.experimental import pallas as pl
from jax.experimental.pallas import tpu as pltpu
from jax.experimental.pallas import tpu_sc as plsc

N_NODES = 10000
N_PAD = 10240
D = 256
NQ = 4
HQ = 64
C = 125
NBUF = 5

NC = 2
NS = 16

_ROWS_PER_TILE = N_PAD // NS
_ZB = 160


def _sc_mesh():
    return plsc.VectorSubcoreMesh(
        core_axis_name="c", subcore_axis_name="s", num_cores=NC, num_subcores=NS
    )


def _deg_body(dst3_hbm, out_hbm, idx_v, ones_v, zero_v, acc, ssem):
    c = lax.axis_index("c")
    s = lax.axis_index("s")
    per_tile = dst3_hbm.shape[1]

    def fill(i, _):
        ones_v[i, :] = jnp.full((16,), 1.0, jnp.float32)
        return 0

    lax.fori_loop(0, C, fill, 0)

    def zfill(i, _):
        zero_v[i, :] = jnp.zeros((16,), jnp.float32)
        return 0

    lax.fori_loop(0, _ROWS_PER_TILE, zfill, 0)

    pltpu.sync_copy(zero_v, acc.at[pl.ds(s * _ROWS_PER_TILE, _ROWS_PER_TILE)])
    pltpu.sync_copy(dst3_hbm.at[s], idx_v)
    plsc.subcore_barrier()

    def body(j, _):
        pltpu.async_copy(ones_v, acc.at[idx_v.at[j]], ssem, add=True)
        return 0

    lax.fori_loop(0, per_tile, body, 0)

    def drain(j, _):
        pltpu.make_async_copy(ones_v, acc.at[idx_v.at[j]], ssem).wait()
        return 0

    lax.fori_loop(0, per_tile, drain, 0)
    plsc.subcore_barrier()

    @pl.when(c == 0)
    def _():
        pltpu.sync_copy(
            acc.at[pl.ds(s * _ROWS_PER_TILE, _ROWS_PER_TILE)],
            out_hbm.at[pl.ds(s * _ROWS_PER_TILE, _ROWS_PER_TILE)],
        )


def _deg_counts(dst3):
    per_tile = dst3.shape[1]
    k = pl.kernel(
        _deg_body,
        out_type=jax.ShapeDtypeStruct((N_PAD, 16), jnp.float32),
        mesh=_sc_mesh(),
        compiler_params=pltpu.CompilerParams(use_tc_tiling_on_sc=False),
        scratch_types=[
            pltpu.VMEM((per_tile, C), jnp.int32),
            pltpu.VMEM((C, 16), jnp.float32),
            pltpu.VMEM((_ROWS_PER_TILE, 16), jnp.float32),
            pltpu.VMEM_SHARED((N_PAD, 16), jnp.float32),
            pltpu.SemaphoreType.DMA,
        ],
    )
    return k(dst3)


def _hop_body(m_hbm, src3_hbm, dst3_hbm, out_hbm, isrc_v, idst_v, rows_v, zb_v, acc, gsem):
    c = lax.axis_index("c")
    s = lax.axis_index("s")
    per_tile = src3_hbm.shape[1]

    def zrow(i, _):
        def zcol(j, _):
            zb_v[i, pl.ds(j * 16, 16)] = jnp.zeros((16,), jnp.float32)
            return 0

        lax.fori_loop(0, HQ // 16, zcol, 0)
        return 0

    lax.fori_loop(0, _ZB, zrow, 0)

    pltpu.sync_copy(src3_hbm.at[s], isrc_v)
    pltpu.sync_copy(dst3_hbm.at[s], idst_v)

    def prime(q):
        for b in range(NBUF):
            pltpu.async_copy(m_hbm.at[q].at[isrc_v.at[b]], rows_v.at[b], gsem.at[b])

    prime(2 * c)

    for p in range(2):
        q = 2 * c + p
        for i in range(_ROWS_PER_TILE // _ZB):
            pltpu.sync_copy(zb_v, acc.at[pl.ds(s * _ROWS_PER_TILE + i * _ZB, _ZB)])
        plsc.subcore_barrier()

        def group(g, _):
            for b in range(NBUF):
                ch = g * NBUF + b
                pltpu.make_async_copy(
                    m_hbm.at[q].at[isrc_v.at[ch]], rows_v.at[b], gsem.at[b]
                ).wait()
                pltpu.sync_copy(rows_v.at[b], acc.at[idst_v.at[ch]], add=True)
                nxt = ch + NBUF

                @pl.when(nxt < per_tile)
                def _():
                    pltpu.async_copy(
                        m_hbm.at[q].at[isrc_v.at[nxt]], rows_v.at[b], gsem.at[b]
                    )
            return 0

        lax.fori_loop(0, per_tile // NBUF, group, 0)
        if p == 0:
            prime(2 * c + 1)
        plsc.subcore_barrier()

        pltpu.sync_copy(
            acc.at[pl.ds(s * _ROWS_PER_TILE, _ROWS_PER_TILE)],
            out_hbm.at[q].at[pl.ds(s * _ROWS_PER_TILE, _ROWS_PER_TILE)],
        )


def _hop(m_q, src3, dst3):
    per_tile = src3.shape[1]
    k = pl.kernel(
        _hop_body,
        out_type=jax.ShapeDtypeStruct((NQ, N_PAD, HQ), jnp.float32),
        mesh=_sc_mesh(),
        compiler_params=pltpu.CompilerParams(use_tc_tiling_on_sc=False),
        scratch_types=[
            pltpu.VMEM((per_tile, C), jnp.int32),
            pltpu.VMEM((per_tile, C), jnp.int32),
            pltpu.VMEM((NBUF, C, HQ), jnp.float32),
            pltpu.VMEM((_ZB, HQ), jnp.float32),
            pltpu.VMEM_SHARED((N_PAD, HQ), jnp.float32),
            pltpu.SemaphoreType.DMA((NBUF,)),
        ],
    )
    return k(m_q, src3, dst3)


_RB = 1000

_BLKQ = pl.BlockSpec((NQ, _RB, HQ), lambda i: (0, i, 0))
_BLKF = pl.BlockSpec((_RB, D), lambda i: (i, 0))


def _split_q(o_ref, res):
    for q in range(NQ):
        o_ref[q] = res[:, q * HQ:(q + 1) * HQ]


def _fc_in_kernel(h_ref, w_ref, b_ref, o_ref, oq_ref):
    res = jnp.dot(h_ref[...], w_ref[...], preferred_element_type=jnp.float32)
    res = res + b_ref[...]
    o_ref[...] = res
    _split_q(oq_ref, res)


def _fc_in(h, w, b):
    return pl.pallas_call(
        _fc_in_kernel,
        grid=(N_NODES // _RB,),
        in_specs=[
            _BLKF,
            pl.BlockSpec((D, D), lambda i: (0, 0)),
            pl.BlockSpec((1, D), lambda i: (0, 0)),
        ],
        out_specs=[_BLKF, _BLKQ],
        out_shape=[
            jax.ShapeDtypeStruct((N_NODES, D), jnp.float32),
            jax.ShapeDtypeStruct((NQ, N_NODES, HQ), jnp.float32),
        ],
    )(h, w, b.reshape(1, D))


def _gru_kernel(x_ref, s_ref, w_ref, u_ref, b_ref, o_ref):
    bf = jnp.bfloat16
    gi = jnp.dot(x_ref[...].astype(bf), w_ref[...].astype(bf),
                 preferred_element_type=jnp.float32)
    gi = gi + b_ref[...]
    gh = jnp.dot(s_ref[...].astype(bf), u_ref[...].astype(bf),
                 preferred_element_type=jnp.float32)
    z = jax.nn.sigmoid(gi[:, :D] + gh[:, :D])
    r = jax.nn.sigmoid(gi[:, D:2 * D] + gh[:, D:2 * D])
    nn = jnp.tanh(gi[:, 2 * D:] + r * gh[:, 2 * D:])
    o_ref[...] = (1.0 - z) * nn + z * s_ref[...]


def _gru_step(x, s, w, u, b):
    return pl.pallas_call(
        _gru_kernel,
        grid=(N_NODES // _RB,),
        in_specs=[
            _BLKF,
            _BLKF,
            pl.BlockSpec((D, 3 * D), lambda i: (0, 0)),
            pl.BlockSpec((D, 3 * D), lambda i: (0, 0)),
            pl.BlockSpec((1, 3 * D), lambda i: (0, 0)),
        ],
        out_specs=_BLKF,
        out_shape=jax.ShapeDtypeStruct((N_NODES, D), jnp.float32),
    )(x, s, w, u, b.reshape(1, 3 * D))


def _gru_first_kernel(x_ref, w_ref, b_ref, o_ref):
    bf = jnp.bfloat16
    gi = jnp.dot(x_ref[...].astype(bf), w_ref[...].astype(bf),
                 preferred_element_type=jnp.float32)
    gi = gi + b_ref[...]
    z = jax.nn.sigmoid(gi[:, :D])
    nn = jnp.tanh(gi[:, 2 * D:])
    o_ref[...] = (1.0 - z) * nn


def _gru_first(x, w, b):
    return pl.pallas_call(
        _gru_first_kernel,
        grid=(N_NODES // _RB,),
        in_specs=[
            _BLKF,
            pl.BlockSpec((D, 3 * D), lambda i: (0, 0)),
            pl.BlockSpec((1, 3 * D), lambda i: (0, 0)),
        ],
        out_specs=_BLKF,
        out_shape=jax.ShapeDtypeStruct((N_NODES, D), jnp.float32),
    )(x, w, b.reshape(1, 3 * D))


def _scale_kernel(agg_ref, d_ref, o_ref, oq_ref):
    inv = 1.0 / jnp.maximum(d_ref[:, 0:1], 1.0)
    parts = [agg_ref[q] * inv for q in range(NQ)]
    o_ref[...] = jnp.concatenate(parts, axis=1)
    for q in range(NQ):
        oq_ref[q] = parts[q]


def _scale(agg, dcnt):
    return pl.pallas_call(
        _scale_kernel,
        grid=(N_NODES // _RB,),
        in_specs=[
            _BLKQ,
            pl.BlockSpec((_RB, 16), lambda i: (i, 0)),
        ],
        out_specs=[_BLKF, _BLKQ],
        out_shape=[
            jax.ShapeDtypeStruct((N_NODES, D), jnp.float32),
            jax.ShapeDtypeStruct((NQ, N_NODES, HQ), jnp.float32),
        ],
    )(agg, dcnt)


def _elu(x):
    return jnp.where(x > 0, x, jnp.exp(x) - 1.0)


def _tail_kernel(s1, s2, s3, s4, h0_ref, dw_ref, db_ref, ym_ref, ymq_ref, loss_ref):
    ys = [_elu(sr[...]) for sr in (s1, s2, s3, s4)]
    ym = (ys[0] + ys[1] + ys[2] + ys[3]) * 0.25
    recon = jnp.dot(ym, dw_ref[...], preferred_element_type=jnp.float32)
    recon = recon + db_ref[...]
    diff = recon - h0_ref[...]

    @pl.when(pl.program_id(0) == 0)
    def _():
        loss_ref[...] = jnp.zeros((1, 1), jnp.float32)

    loss_ref[...] += jnp.sum(diff * diff).reshape(1, 1)
    ym_ref[...] = ym
    _split_q(ymq_ref, ym)


def _layer_tail(s_list, h0, dec_w, dec_b):
    return pl.pallas_call(
        _tail_kernel,
        grid=(N_NODES // _RB,),
        in_specs=[
            _BLKF, _BLKF, _BLKF, _BLKF,
            _BLKF,
            pl.BlockSpec((D, D), lambda i: (0, 0)),
            pl.BlockSpec((1, D), lambda i: (0, 0)),
        ],
        out_specs=[
            _BLKF,
            _BLKQ,
            pl.BlockSpec((1, 1), lambda i: (0, 0)),
        ],
        out_shape=[
            jax.ShapeDtypeStruct((N_NODES, D), jnp.float32),
            jax.ShapeDtypeStruct((NQ, N_NODES, HQ), jnp.float32),
            jax.ShapeDtypeStruct((1, 1), jnp.float32),
        ],
    )(*s_list, h0, dec_w, dec_b.reshape(1, D))


def _fc_out_kernel(s1, s2, s3, s4, w_ref, b_ref, o_ref):
    for t, sr in enumerate((s1, s2, s3, s4)):
        y = _elu(sr[...])
        logits = jnp.dot(y, w_ref[...], preferred_element_type=jnp.float32)
        logits = logits + b_ref[...]
        m = jnp.max(logits, axis=1, keepdims=True)
        p = jnp.exp(logits - m)
        o_ref[t] = p / jnp.sum(p, axis=1, keepdims=True)


def _fc_out(s_list, w, b):
    dout = w.shape[1]
    return pl.pallas_call(
        _fc_out_kernel,
        grid=(N_NODES // _RB,),
        in_specs=[
            _BLKF, _BLKF, _BLKF, _BLKF,
            pl.BlockSpec((D, dout), lambda i: (0, 0)),
            pl.BlockSpec((1, dout), lambda i: (0, 0)),
        ],
        out_specs=pl.BlockSpec((4, _RB, dout), lambda i: (0, i, 0)),
        out_shape=jax.ShapeDtypeStruct((4, N_NODES, dout), jnp.float32),
    )(*s_list, w, b.reshape(1, dout))


def kernel(h, edge_index, params):
    n, d_in = h.shape
    e = edge_index.shape[1]
    assert n == N_NODES and d_in == D and e % (C * NS) == 0 and (e // (C * NS)) % NBUF == 0

    src3 = edge_index[0].reshape(NS, e // C // NS, C).astype(jnp.int32)
    dst3 = edge_index[1].reshape(NS, e // C // NS, C).astype(jnp.int32)

    dcnt = _deg_counts(dst3)
    x, xq = _fc_in(h, params['fc_in_w'], params['fc_in_b'])

    loss = jnp.float32(0.0)
    out = None
    walk_len = 4
    for li, p in enumerate(params['layers']):
        mq = xq
        states = []
        s = _gru_first(x, p['W'], p['b'])
        for _ in range(walk_len):
            agg = _hop(mq, src3, dst3)
            m, mq = _scale(agg, dcnt)
            s = _gru_step(m, s, p['W'], p['U'], p['b'])
            states.append(s)
        ymean, ymq, losssum = _layer_tail(states, h, p['dec_w'], p['dec_b'])
        loss = loss + 0.05 * (losssum[0, 0] / (n * d_in))
        x, xq = ymean, ymq
        if li == len(params['layers']) - 1:
            out = _fc_out(states, params['fc_out_w'], params['fc_out_b'])
    return out, loss


# speedup vs baseline: 1.1112x; 1.0008x over previous
import jax
import jax.numpy as jnp
from jax import lax
from jax.experimental import pallas as pl
from jax.experimental.pallas import tpu as pltpu
from jax.experimental.pallas import tpu_sc as plsc

N_NODES = 10000
N_PAD = 10240
D = 256
NQ = 4
HQ = 64
C = 125
NBUF = 5

NC = 2
NS = 16

_ROWS_PER_TILE = N_PAD // NS
_ZB = 160


def _sc_mesh():
    return plsc.VectorSubcoreMesh(
        core_axis_name="c", subcore_axis_name="s", num_cores=NC, num_subcores=NS
    )


def _deg_body(dst3_hbm, out_hbm, idx_v, ones_v, zero_v, acc, ssem):
    c = lax.axis_index("c")
    s = lax.axis_index("s")
    per_tile = dst3_hbm.shape[1]

    def fill(i, _):
        ones_v[i, :] = jnp.full((16,), 1.0, jnp.float32)
        return 0

    lax.fori_loop(0, C, fill, 0)

    def zfill(i, _):
        zero_v[i, :] = jnp.zeros((16,), jnp.float32)
        return 0

    lax.fori_loop(0, _ROWS_PER_TILE, zfill, 0)

    pltpu.sync_copy(zero_v, acc.at[pl.ds(s * _ROWS_PER_TILE, _ROWS_PER_TILE)])
    pltpu.sync_copy(dst3_hbm.at[s], idx_v)
    plsc.subcore_barrier()

    def body(j, _):
        pltpu.async_copy(ones_v, acc.at[idx_v.at[j]], ssem, add=True)
        return 0

    lax.fori_loop(0, per_tile, body, 0)

    def drain(j, _):
        pltpu.make_async_copy(ones_v, acc.at[idx_v.at[j]], ssem).wait()
        return 0

    lax.fori_loop(0, per_tile, drain, 0)
    plsc.subcore_barrier()

    @pl.when(c == 0)
    def _():
        pltpu.sync_copy(
            acc.at[pl.ds(s * _ROWS_PER_TILE, _ROWS_PER_TILE)],
            out_hbm.at[pl.ds(s * _ROWS_PER_TILE, _ROWS_PER_TILE)],
        )


def _deg_counts(dst3):
    per_tile = dst3.shape[1]
    k = pl.kernel(
        _deg_body,
        out_type=jax.ShapeDtypeStruct((N_PAD, 16), jnp.float32),
        mesh=_sc_mesh(),
        compiler_params=pltpu.CompilerParams(use_tc_tiling_on_sc=False),
        scratch_types=[
            pltpu.VMEM((per_tile, C), jnp.int32),
            pltpu.VMEM((C, 16), jnp.float32),
            pltpu.VMEM((_ROWS_PER_TILE, 16), jnp.float32),
            pltpu.VMEM_SHARED((N_PAD, 16), jnp.float32),
            pltpu.SemaphoreType.DMA,
        ],
    )
    return k(dst3)


def _hop_body(m_hbm, src3_hbm, dst3_hbm, out_hbm, isrc_v, idst_v, rows_v, zb_v, acc, gsem):
    c = lax.axis_index("c")
    s = lax.axis_index("s")
    per_tile = src3_hbm.shape[1]

    def zrow(i, _):
        def zcol(j, _):
            zb_v[i, pl.ds(j * 16, 16)] = jnp.zeros((16,), jnp.float32)
            return 0

        lax.fori_loop(0, HQ // 16, zcol, 0)
        return 0

    lax.fori_loop(0, _ZB, zrow, 0)

    pltpu.sync_copy(src3_hbm.at[s], isrc_v)
    pltpu.sync_copy(dst3_hbm.at[s], idst_v)

    def prime(q):
        for b in range(NBUF):
            pltpu.async_copy(m_hbm.at[q].at[isrc_v.at[b]], rows_v.at[b], gsem.at[b])

    prime(2 * c)

    for p in range(2):
        q = 2 * c + p
        for i in range(_ROWS_PER_TILE // _ZB):
            pltpu.sync_copy(zb_v, acc.at[pl.ds(s * _ROWS_PER_TILE + i * _ZB, _ZB)])
        plsc.subcore_barrier()

        def group(g, _):
            for b in range(NBUF):
                ch = g * NBUF + b
                pltpu.make_async_copy(
                    m_hbm.at[q].at[isrc_v.at[ch]], rows_v.at[b], gsem.at[b]
                ).wait()
                pltpu.sync_copy(rows_v.at[b], acc.at[idst_v.at[ch]], add=True)
                nxt = ch + NBUF

                @pl.when(nxt < per_tile)
                def _():
                    pltpu.async_copy(
                        m_hbm.at[q].at[isrc_v.at[nxt]], rows_v.at[b], gsem.at[b]
                    )
            return 0

        lax.fori_loop(0, per_tile // NBUF, group, 0)
        if p == 0:
            prime(2 * c + 1)
        plsc.subcore_barrier()

        pltpu.sync_copy(
            acc.at[pl.ds(s * _ROWS_PER_TILE, _ROWS_PER_TILE)],
            out_hbm.at[q].at[pl.ds(s * _ROWS_PER_TILE, _ROWS_PER_TILE)],
        )


def _hop(m_q, src3, dst3):
    per_tile = src3.shape[1]
    k = pl.kernel(
        _hop_body,
        out_type=jax.ShapeDtypeStruct((NQ, N_PAD, HQ), jnp.float32),
        mesh=_sc_mesh(),
        compiler_params=pltpu.CompilerParams(use_tc_tiling_on_sc=False),
        scratch_types=[
            pltpu.VMEM((per_tile, C), jnp.int32),
            pltpu.VMEM((per_tile, C), jnp.int32),
            pltpu.VMEM((NBUF, C, HQ), jnp.float32),
            pltpu.VMEM((_ZB, HQ), jnp.float32),
            pltpu.VMEM_SHARED((N_PAD, HQ), jnp.float32),
            pltpu.SemaphoreType.DMA((NBUF,)),
        ],
    )
    return k(m_q, src3, dst3)


_RB = 1000

_BLKQ = pl.BlockSpec((NQ, _RB, HQ), lambda i: (0, i, 0))
_BLKF = pl.BlockSpec((_RB, D), lambda i: (i, 0))


def _split_q(o_ref, res):
    for q in range(NQ):
        o_ref[q] = res[:, q * HQ:(q + 1) * HQ]


def _fc_in_kernel(h_ref, w_ref, b_ref, o_ref, oq_ref):
    res = jnp.dot(h_ref[...], w_ref[...], preferred_element_type=jnp.float32)
    res = res + b_ref[...]
    o_ref[...] = res
    _split_q(oq_ref, res)


def _fc_in(h, w, b):
    return pl.pallas_call(
        _fc_in_kernel,
        grid=(N_NODES // _RB,),
        in_specs=[
            _BLKF,
            pl.BlockSpec((D, D), lambda i: (0, 0)),
            pl.BlockSpec((1, D), lambda i: (0, 0)),
        ],
        out_specs=[_BLKF, _BLKQ],
        out_shape=[
            jax.ShapeDtypeStruct((N_NODES, D), jnp.float32),
            jax.ShapeDtypeStruct((NQ, N_NODES, HQ), jnp.float32),
        ],
    )(h, w, b.reshape(1, D))


def _gru_kernel(x_ref, s_ref, w_ref, u_ref, b_ref, o_ref):
    gi = jnp.dot(x_ref[...], w_ref[...], preferred_element_type=jnp.float32)
    gi = gi + b_ref[...]
    gh = jnp.dot(s_ref[...], u_ref[...], preferred_element_type=jnp.float32)
    z = jax.nn.sigmoid(gi[:, :D] + gh[:, :D])
    r = jax.nn.sigmoid(gi[:, D:2 * D] + gh[:, D:2 * D])
    nn = jnp.tanh(gi[:, 2 * D:] + r * gh[:, 2 * D:])
    o_ref[...] = (1.0 - z) * nn + z * s_ref[...]


def _gru_step(x, s, w, u, b):
    return pl.pallas_call(
        _gru_kernel,
        grid=(N_NODES // _RB,),
        in_specs=[
            _BLKF,
            _BLKF,
            pl.BlockSpec((D, 3 * D), lambda i: (0, 0)),
            pl.BlockSpec((D, 3 * D), lambda i: (0, 0)),
            pl.BlockSpec((1, 3 * D), lambda i: (0, 0)),
        ],
        out_specs=_BLKF,
        out_shape=jax.ShapeDtypeStruct((N_NODES, D), jnp.float32),
    )(x, s, w, u, b.reshape(1, 3 * D))


def _gru_first_kernel(x_ref, w_ref, b_ref, o_ref):
    gi = jnp.dot(x_ref[...], w_ref[...], preferred_element_type=jnp.float32)
    gi = gi + b_ref[...]
    z = jax.nn.sigmoid(gi[:, :D])
    nn = jnp.tanh(gi[:, 2 * D:])
    o_ref[...] = (1.0 - z) * nn


def _gru_first(x, w, b):
    return pl.pallas_call(
        _gru_first_kernel,
        grid=(N_NODES // _RB,),
        in_specs=[
            _BLKF,
            pl.BlockSpec((D, 3 * D), lambda i: (0, 0)),
            pl.BlockSpec((1, 3 * D), lambda i: (0, 0)),
        ],
        out_specs=_BLKF,
        out_shape=jax.ShapeDtypeStruct((N_NODES, D), jnp.float32),
    )(x, w, b.reshape(1, 3 * D))


def _scale_kernel(agg_ref, d_ref, o_ref, oq_ref):
    inv = 1.0 / jnp.maximum(d_ref[:, 0:1], 1.0)
    parts = [agg_ref[q] * inv for q in range(NQ)]
    o_ref[...] = jnp.concatenate(parts, axis=1)
    for q in range(NQ):
        oq_ref[q] = parts[q]


def _scale(agg, dcnt):
    return pl.pallas_call(
        _scale_kernel,
        grid=(N_NODES // _RB,),
        in_specs=[
            _BLKQ,
            pl.BlockSpec((_RB, 16), lambda i: (i, 0)),
        ],
        out_specs=[_BLKF, _BLKQ],
        out_shape=[
            jax.ShapeDtypeStruct((N_NODES, D), jnp.float32),
            jax.ShapeDtypeStruct((NQ, N_NODES, HQ), jnp.float32),
        ],
    )(agg, dcnt)


def _elu(x):
    return jnp.where(x > 0, x, jnp.exp(x) - 1.0)


def _tail_kernel(s1, s2, s3, s4, h0_ref, dw_ref, db_ref, ym_ref, ymq_ref, loss_ref):
    ys = [_elu(sr[...]) for sr in (s1, s2, s3, s4)]
    ym = (ys[0] + ys[1] + ys[2] + ys[3]) * 0.25
    recon = jnp.dot(ym, dw_ref[...], preferred_element_type=jnp.float32)
    recon = recon + db_ref[...]
    diff = recon - h0_ref[...]

    @pl.when(pl.program_id(0) == 0)
    def _():
        loss_ref[...] = jnp.zeros((1, 1), jnp.float32)

    loss_ref[...] += jnp.sum(diff * diff).reshape(1, 1)
    ym_ref[...] = ym
    _split_q(ymq_ref, ym)


def _layer_tail(s_list, h0, dec_w, dec_b):
    return pl.pallas_call(
        _tail_kernel,
        grid=(N_NODES // _RB,),
        in_specs=[
            _BLKF, _BLKF, _BLKF, _BLKF,
            _BLKF,
            pl.BlockSpec((D, D), lambda i: (0, 0)),
            pl.BlockSpec((1, D), lambda i: (0, 0)),
        ],
        out_specs=[
            _BLKF,
            _BLKQ,
            pl.BlockSpec((1, 1), lambda i: (0, 0)),
        ],
        out_shape=[
            jax.ShapeDtypeStruct((N_NODES, D), jnp.float32),
            jax.ShapeDtypeStruct((NQ, N_NODES, HQ), jnp.float32),
            jax.ShapeDtypeStruct((1, 1), jnp.float32),
        ],
    )(*s_list, h0, dec_w, dec_b.reshape(1, D))


def _fc_out_kernel(s1, s2, s3, s4, w_ref, b_ref, o_ref):
    for t, sr in enumerate((s1, s2, s3, s4)):
        y = _elu(sr[...])
        logits = jnp.dot(y, w_ref[...], preferred_element_type=jnp.float32)
        logits = logits + b_ref[...]
        m = jnp.max(logits, axis=1, keepdims=True)
        p = jnp.exp(logits - m)
        o_ref[t] = p / jnp.sum(p, axis=1, keepdims=True)


def _fc_out(s_list, w, b):
    dout = w.shape[1]
    return pl.pallas_call(
        _fc_out_kernel,
        grid=(N_NODES // _RB,),
        in_specs=[
            _BLKF, _BLKF, _BLKF, _BLKF,
            pl.BlockSpec((D, dout), lambda i: (0, 0)),
            pl.BlockSpec((1, dout), lambda i: (0, 0)),
        ],
        out_specs=pl.BlockSpec((4, _RB, dout), lambda i: (0, i, 0)),
        out_shape=jax.ShapeDtypeStruct((4, N_NODES, dout), jnp.float32),
    )(*s_list, w, b.reshape(1, dout))


def kernel(h, edge_index, params):
    n, d_in = h.shape
    e = edge_index.shape[1]
    assert n == N_NODES and d_in == D and e % (C * NS) == 0 and (e // (C * NS)) % NBUF == 0

    src3 = edge_index[0].reshape(NS, e // C // NS, C).astype(jnp.int32)
    dst3 = edge_index[1].reshape(NS, e // C // NS, C).astype(jnp.int32)

    dcnt = _deg_counts(dst3)
    x, xq = _fc_in(h, params['fc_in_w'], params['fc_in_b'])

    loss = jnp.float32(0.0)
    out = None
    walk_len = 4
    for li, p in enumerate(params['layers']):
        mq = xq
        states = []
        s = _gru_first(x, p['W'], p['b'])
        for _ in range(walk_len):
            agg = _hop(mq, src3, dst3)
            m, mq = _scale(agg, dcnt)
            s = _gru_step(m, s, p['W'], p['U'], p['b'])
            states.append(s)
        ymean, ymq, losssum = _layer_tail(states, h, p['dec_w'], p['dec_b'])
        loss = loss + 0.05 * (losssum[0, 0] / (n * d_in))
        x, xq = ymean, ymq
        if li == len(params['layers']) - 1:
            out = _fc_out(states, params['fc_out_w'], params['fc_out_b'])
    return out, loss
